# baseline (speedup 1.0000x reference)
import functools

import jax
import jax.numpy as jnp
from jax import lax
from jax.experimental import pallas as pl
from jax.experimental.pallas import tpu as pltpu
from jax.experimental.pallas import tpu_sc as plsc

NWORDS = 1000000
NTAGS = 16
SEQ = 16384

NC = 2
NS = 16
NW = NC * NS
BPW = SEQ // NW
CW = 128
CHUNKS = BPW // CW


def _bow_body(table_hbm, xidx_hbm, out_hbm, idx_v, rows_v, stage_v, sem):
    c = lax.axis_index("c")
    s = lax.axis_index("s")
    wid = s * NC + c
    pltpu.sync_copy(xidx_hbm.at[pl.ds(wid * CHUNKS, CHUNKS)], idx_v)
    handles = [
        pltpu.async_copy(
            table_hbm.at[idx_v.at[j]],
            rows_v.at[pl.ds(j * CW, CW)],
            sem,
        )
        for j in range(CHUNKS)
    ]
    for h in handles:
        h.wait()

    def body(i, acc):
        return acc + rows_v[i]

    acc = lax.fori_loop(0, BPW, body, jnp.zeros((NTAGS,), jnp.float32))
    stage_v[...] = acc
    pltpu.sync_copy(stage_v, out_hbm.at[wid])


_bow_sc = functools.partial(
    pl.kernel,
    out_type=jax.ShapeDtypeStruct((NW, NTAGS), jnp.float32),
    mesh=plsc.VectorSubcoreMesh(core_axis_name="c", subcore_axis_name="s"),
    scratch_types=[
        pltpu.VMEM((CHUNKS, CW), jnp.int32),
        pltpu.VMEM((BPW, NTAGS), jnp.float32),
        pltpu.VMEM((NTAGS,), jnp.float32),
        pltpu.SemaphoreType.DMA,
    ],
    compiler_params=pltpu.CompilerParams(use_tc_tiling_on_sc=False),
)(_bow_body)


def kernel(x, table, bias):
    xi = x.astype(jnp.int32).reshape(NW * CHUNKS, CW)
    partials = _bow_sc(table, xi)
    return (jnp.sum(partials, axis=0) + bias).reshape(1, -1)
